# baseline (speedup 1.0000x reference)
import functools

import jax
import jax.numpy as jnp
from jax import lax
from jax.experimental import pallas as pl
from jax.experimental.pallas import tpu as pltpu
from jax.experimental.pallas import tpu_sc as plsc

D_MODEL = 64
SCALE = float(D_MODEL) ** 0.5
LANES = 16

NC = 2
NS = 16
NW = NC * NS

B = 4096
S = 200
BPW = B // NW
SPLIT = (96, 104)
NG = 2 * BPW
NB = 8
LEAD = NB // 2
NH = NG // NB

_mesh = plsc.VectorSubcoreMesh(core_axis_name="c", subcore_axis_name="s")


@functools.partial(
    pl.kernel,
    mesh=_mesh,
    out_type=jax.ShapeDtypeStruct((B, S, D_MODEL), jnp.float32),
    scratch_types=(
        [pltpu.VMEM((BPW, S), jnp.int32)]
        + [pltpu.VMEM((SPLIT[i % 2], D_MODEL), jnp.float32) for i in range(NB)]
        + [pltpu.SemaphoreType.DMA for _ in range(2 * NB)]
    ),
    compiler_params=pltpu.CompilerParams(use_tc_tiling_on_sc=False),
)
def _embed(idx_hbm, table_hbm, out_hbm, idx_v, *bufs):
    rows = bufs[0:NB]
    gsem = bufs[NB:2 * NB]
    osem = bufs[2 * NB:3 * NB]
    wid = lax.axis_index("s") * NC + lax.axis_index("c")
    pltpu.sync_copy(idx_hbm.at[wid], idx_v)
    row_base = wid * BPW

    def src_dst(g, b):
        half = b % 2
        L = SPLIT[half]
        r = g // 2
        idx_view = idx_v.at[r, pl.ds(96 * half, L)]
        out_view = out_hbm.at[row_base + r, pl.ds(96 * half, L)]
        return idx_view, out_view, L

    def gather_start(g, b):
        idx_view, _, _ = src_dst(g, b)
        pltpu.async_copy(table_hbm.at[idx_view], rows[b], gsem[b])

    def gather_wait(g, b):
        idx_view, _, _ = src_dst(g, b)
        pltpu.make_async_copy(table_hbm.at[idx_view], rows[b], gsem[b]).wait()

    def scatter_start(g, b):
        _, out_view, _ = src_dst(g, b)
        pltpu.async_copy(rows[b], out_view, osem[b])

    def scatter_wait(b):
        L = SPLIT[b % 2]
        pltpu.make_async_copy(
            rows[b], out_hbm.at[row_base, pl.ds(96 * (b % 2), L)], osem[b]
        ).wait()

    def visit(g, b, pre_fetch, pre_wait):
        b_pre = (b + LEAD) % NB
        L = SPLIT[b % 2]
        gather_wait(g, b)

        def scale_row(r, c2):
            for c in range(D_MODEL // LANES):
                sl = pl.ds(c * LANES, LANES)
                rows[b][r, sl] = rows[b][r, sl] * SCALE
            return c2

        lax.fori_loop(0, L, scale_row, 0, unroll=4)
        scatter_start(g, b)
        if pre_fetch:
            if pre_wait:
                scatter_wait(b_pre)
            gather_start(g + LEAD, b_pre)

    for b in range(LEAD):
        gather_start(b, b)

    for b in range(NB):
        visit(b, b, pre_fetch=True, pre_wait=(b >= LEAD))

    def outer(h, carry):
        for b in range(NB):
            visit(h * NB + b, b, pre_fetch=True, pre_wait=True)
        return carry

    lax.fori_loop(1, NH - 1, outer, 0)

    for b in range(NB):
        visit((NH - 1) * NB + b, b, pre_fetch=(b < LEAD), pre_wait=True)

    for b in range(LEAD, NB):
        scatter_wait(b)


def kernel(x, table):
    idx = x.reshape(NW, BPW, S)
    return _embed(idx, table)


# speedup vs baseline: 1.2183x; 1.2183x over previous
import functools

import jax
import jax.numpy as jnp
from jax import lax
from jax.experimental import pallas as pl
from jax.experimental.pallas import tpu as pltpu
from jax.experimental.pallas import tpu_sc as plsc

D_MODEL = 64
DP = 128
SCALE = float(D_MODEL) ** 0.5
LANES = 16

NC, NS = 2, 16
NW = NC * NS

ROWS = 4096 * 200
RPW = ROWS // NW
G = 64
NG = RPW // G
NB = 8
LEAD = NB // 2
NH = NG // NB

_mesh = plsc.VectorSubcoreMesh(core_axis_name="c", subcore_axis_name="s")


@functools.partial(
    pl.kernel,
    mesh=_mesh,
    out_type=jax.ShapeDtypeStruct((ROWS, DP), jnp.float32),
    scratch_types=(
        [pltpu.VMEM((NG, G), jnp.int32)]
        + [pltpu.VMEM((G, DP), jnp.float32) for _ in range(NB)]
        + [pltpu.SemaphoreType.DMA for _ in range(2 * NB)]
    ),
    compiler_params=pltpu.CompilerParams(use_tc_tiling_on_sc=True),
)
def _embed(idx_hbm, table_hbm, out_hbm, idx_v, *bufs):
    rows = bufs[0:NB]
    gsem = bufs[NB:2 * NB]
    osem = bufs[2 * NB:3 * NB]
    wid = lax.axis_index("s") * NC + lax.axis_index("c")
    pltpu.sync_copy(idx_hbm.at[wid], idx_v)
    out_base = wid * RPW

    def gather_start(g, b):
        pltpu.async_copy(table_hbm.at[idx_v.at[g]], rows[b], gsem[b])

    def gather_wait(g, b):
        pltpu.make_async_copy(table_hbm.at[idx_v.at[g]], rows[b], gsem[b]).wait()

    def scatter_start(g, b):
        pltpu.async_copy(rows[b], out_hbm.at[pl.ds(out_base + g * G, G)], osem[b])

    def scatter_wait(b):
        pltpu.make_async_copy(
            rows[b], out_hbm.at[pl.ds(out_base, G)], osem[b]).wait()

    def visit(g, b, pre_fetch, pre_wait):
        b_pre = (b + LEAD) % NB
        gather_wait(g, b)

        def scale_row(r, c2):
            for c in range(D_MODEL // LANES):
                sl = pl.ds(c * LANES, LANES)
                rows[b][r, sl] = rows[b][r, sl] * SCALE
            return c2

        lax.fori_loop(0, G, scale_row, 0, unroll=4)
        scatter_start(g, b)
        if pre_fetch:
            if pre_wait:
                scatter_wait(b_pre)
            gather_start(g + LEAD, b_pre)

    for b in range(LEAD):
        gather_start(b, b)
    for b in range(NB):
        visit(b, b, pre_fetch=True, pre_wait=(b >= LEAD))

    def outer(h, carry):
        for b in range(NB):
            visit(h * NB + b, b, pre_fetch=True, pre_wait=True)
        return carry

    lax.fori_loop(1, NH - 1, outer, 0)

    for b in range(NB):
        visit((NH - 1) * NB + b, b, pre_fetch=(b < LEAD), pre_wait=True)
    for b in range(LEAD, NB):
        scatter_wait(b)


def kernel(x, table):
    table128 = jnp.pad(table, ((0, 0), (0, DP - D_MODEL)))
    idx = x.reshape(NW, NG, G)
    out = _embed(idx, table128)
    return out.reshape(4096, 200, DP)[:, :, :D_MODEL]
